# baseline (speedup 1.0000x reference)
import jax
import jax.numpy as jnp
from jax import lax
from jax.experimental import pallas as pl

T = 1024
D = 1024
E = 8
FF = 512
NC = 2
FC = FF // NC


def _combine_col(x, wr, e):
    logits = lax.dot_general(x, wr, (((1,), (1,)), ((), ())),
                             preferred_element_type=jnp.float32)
    probs = jax.nn.softmax(logits, axis=-1)
    col = lax.broadcasted_iota(jnp.int32, probs.shape, 1)
    v1 = jnp.max(probs, axis=-1, keepdims=True)
    i1 = jnp.min(jnp.where(probs == v1, col, E), axis=-1, keepdims=True)
    masked = jnp.where(col == i1, -jnp.inf, probs)
    v2 = jnp.max(masked, axis=-1, keepdims=True)
    i2 = jnp.min(jnp.where(masked == v2, col, E), axis=-1, keepdims=True)
    s = v1 + v2
    w1 = v1 / s
    w2 = v2 / s
    return jnp.where(i1 == e, w1, 0.0) + jnp.where(i2 == e, w2, 0.0)


def _moe_body(x_ref, wr_ref, wg_ref, wu_ref, wd_ref, out_ref, c_ref):
    e = pl.program_id(0)
    h = pl.program_id(1)
    x = x_ref[...]

    @pl.when(h == 0)
    def _():
        c_ref[:, 0:1] = _combine_col(x, wr_ref[...], e)

    c_e = c_ref[:, 0:1]
    gate = lax.dot_general(x, wg_ref[0], (((1,), (1,)), ((), ())),
                           preferred_element_type=jnp.float32)
    up = lax.dot_general(x, wu_ref[0], (((1,), (1,)), ((), ())),
                         preferred_element_type=jnp.float32)
    act = gate * jax.nn.sigmoid(gate) * up
    y = lax.dot_general(act, wd_ref[0], (((1,), (1,)), ((), ())),
                        preferred_element_type=jnp.float32)
    contrib = c_e * y

    @pl.when(jnp.logical_and(e == 0, h == 0))
    def _():
        out_ref[...] = contrib

    @pl.when(jnp.logical_or(e != 0, h != 0))
    def _():
        out_ref[...] = out_ref[...] + contrib


def kernel(hidden_states, router_weight, w_gate_up, w_down):
    from jax.experimental.pallas import tpu as pltpu
    return pl.pallas_call(
        _moe_body,
        grid=(E, NC),
        in_specs=[
            pl.BlockSpec((T, D), lambda e, h: (0, 0)),
            pl.BlockSpec((E, D), lambda e, h: (0, 0)),
            pl.BlockSpec((1, FC, D), lambda e, h: (e, h, 0)),
            pl.BlockSpec((1, FC, D), lambda e, h: (e, h + NC, 0)),
            pl.BlockSpec((1, D, FC), lambda e, h: (e, 0, h)),
        ],
        out_specs=pl.BlockSpec((T, D), lambda e, h: (0, 0)),
        out_shape=jax.ShapeDtypeStruct((T, D), jnp.float32),
        scratch_shapes=[pltpu.VMEM((T, 128), jnp.float32)],
    )(hidden_states, router_weight, w_gate_up, w_gate_up, w_down)


# speedup vs baseline: 1.3971x; 1.3971x over previous
import jax
import jax.numpy as jnp
from jax import lax
from jax.experimental import pallas as pl

T = 1024
D = 1024
E = 8
FF = 512


def _combine_all(x, wr):
    logits = lax.dot_general(x, wr, (((1,), (1,)), ((), ())),
                             preferred_element_type=jnp.float32)
    probs = jax.nn.softmax(logits, axis=-1)
    col = lax.broadcasted_iota(jnp.int32, probs.shape, 1)
    v1 = jnp.max(probs, axis=-1, keepdims=True)
    i1 = jnp.min(jnp.where(probs == v1, col, E), axis=-1, keepdims=True)
    masked = jnp.where(col == i1, -jnp.inf, probs)
    v2 = jnp.max(masked, axis=-1, keepdims=True)
    i2 = jnp.min(jnp.where(masked == v2, col, E), axis=-1, keepdims=True)
    s = v1 + v2
    w1 = v1 / s
    w2 = v2 / s
    return jnp.where(col == i1, w1, 0.0) + jnp.where(col == i2, w2, 0.0)


def _moe_body(x_ref, wr_ref, wgu_ref, wd_ref, out_ref, c_ref):
    e = pl.program_id(0)
    x = x_ref[...]

    @pl.when(e == 0)
    def _():
        c_ref[...] = _combine_all(x, wr_ref[...])

    col = lax.broadcasted_iota(jnp.int32, (T, E), 1)
    c_e = jnp.sum(jnp.where(col == e, c_ref[...], 0.0), axis=1, keepdims=True)
    wgu = wgu_ref[0]
    gu = lax.dot_general(x, wgu, (((1,), (1,)), ((), ())),
                         preferred_element_type=jnp.float32)
    gate = gu[:, :FF]
    up = gu[:, FF:]
    act = (gate * jax.nn.sigmoid(gate) * up) * c_e
    wd = wd_ref[0]
    contrib = lax.dot_general(act, wd, (((1,), (1,)), ((), ())),
                              preferred_element_type=jnp.float32)

    @pl.when(e == 0)
    def _():
        out_ref[...] = contrib

    @pl.when(e != 0)
    def _():
        out_ref[...] = out_ref[...] + contrib


def kernel(hidden_states, router_weight, w_gate_up, w_down):
    from jax.experimental.pallas import tpu as pltpu
    return pl.pallas_call(
        _moe_body,
        grid=(E,),
        in_specs=[
            pl.BlockSpec((T, D), lambda e: (0, 0)),
            pl.BlockSpec((E, D), lambda e: (0, 0)),
            pl.BlockSpec((1, 2 * FF, D), lambda e: (e, 0, 0)),
            pl.BlockSpec((1, D, FF), lambda e: (e, 0, 0)),
        ],
        out_specs=pl.BlockSpec((T, D), lambda e: (0, 0)),
        out_shape=jax.ShapeDtypeStruct((T, D), jnp.float32),
        scratch_shapes=[pltpu.VMEM((T, E), jnp.float32)],
    )(hidden_states, router_weight, w_gate_up, w_down)


# speedup vs baseline: 1.4633x; 1.0474x over previous
import jax
import jax.numpy as jnp
from jax import lax
from jax.experimental import pallas as pl

T = 1024
D = 1024
E = 8
FF = 512


def _combine_all(x, wr):
    logits = lax.dot_general(x, wr, (((1,), (1,)), ((), ())),
                             preferred_element_type=jnp.float32)
    probs = jax.nn.softmax(logits, axis=-1)
    col = lax.broadcasted_iota(jnp.int32, probs.shape, 1)
    v1 = jnp.max(probs, axis=-1, keepdims=True)
    i1 = jnp.min(jnp.where(probs == v1, col, E), axis=-1, keepdims=True)
    masked = jnp.where(col == i1, -jnp.inf, probs)
    v2 = jnp.max(masked, axis=-1, keepdims=True)
    i2 = jnp.min(jnp.where(masked == v2, col, E), axis=-1, keepdims=True)
    s = v1 + v2
    w1 = v1 / s
    w2 = v2 / s
    return jnp.where(col == i1, w1, 0.0) + jnp.where(col == i2, w2, 0.0)


def _moe_body(x_ref, wr_ref, wgu_ref, wd_ref, out_ref, c_ref, xb_ref):
    e = pl.program_id(0)

    @pl.when(e == 0)
    def _():
        x = x_ref[...]
        c_ref[...] = _combine_all(x, wr_ref[...])
        xb_ref[...] = x.astype(jnp.bfloat16)

    col = lax.broadcasted_iota(jnp.int32, (T, E), 1)
    c_e = jnp.sum(jnp.where(col == e, c_ref[...], 0.0), axis=1, keepdims=True)
    xb = xb_ref[...]
    wgu = wgu_ref[0].astype(jnp.bfloat16)
    gu = lax.dot_general(xb, wgu, (((1,), (1,)), ((), ())),
                         preferred_element_type=jnp.float32)
    gate = gu[:, :FF]
    up = gu[:, FF:]
    act = (gate * jax.nn.sigmoid(gate) * up) * c_e
    wd = wd_ref[0].astype(jnp.bfloat16)
    contrib = lax.dot_general(act.astype(jnp.bfloat16), wd,
                              (((1,), (1,)), ((), ())),
                              preferred_element_type=jnp.float32)

    @pl.when(e == 0)
    def _():
        out_ref[...] = contrib

    @pl.when(e != 0)
    def _():
        out_ref[...] = out_ref[...] + contrib


def kernel(hidden_states, router_weight, w_gate_up, w_down):
    from jax.experimental.pallas import tpu as pltpu
    return pl.pallas_call(
        _moe_body,
        grid=(E,),
        in_specs=[
            pl.BlockSpec((T, D), lambda e: (0, 0)),
            pl.BlockSpec((E, D), lambda e: (0, 0)),
            pl.BlockSpec((1, 2 * FF, D), lambda e: (e, 0, 0)),
            pl.BlockSpec((1, D, FF), lambda e: (e, 0, 0)),
        ],
        out_specs=pl.BlockSpec((T, D), lambda e: (0, 0)),
        out_shape=jax.ShapeDtypeStruct((T, D), jnp.float32),
        scratch_shapes=[pltpu.VMEM((T, E), jnp.float32),
                        pltpu.VMEM((T, D), jnp.bfloat16)],
    )(hidden_states, router_weight, w_gate_up, w_down)
